# baseline (speedup 1.0000x reference)
import jax
import jax.numpy as jnp
from jax.experimental import pallas as pl
from jax.experimental.pallas import tpu as pltpu

_BM = 400


def _gcn_block(a_ref, e_ref, o_ref):
    o_ref[...] = jnp.dot(a_ref[...], e_ref[...],
                         preferred_element_type=jnp.float32)


def kernel(A_hat, E):
    n, k = A_hat.shape
    d = E.shape[1]
    bm = _BM if n % _BM == 0 else n
    return pl.pallas_call(
        _gcn_block,
        grid=(n // bm,),
        in_specs=[
            pl.BlockSpec((bm, k), lambda i: (i, 0)),
            pl.BlockSpec((k, d), lambda i: (0, 0)),
        ],
        out_specs=pl.BlockSpec((bm, d), lambda i: (i, 0)),
        out_shape=jax.ShapeDtypeStruct((n, d), jnp.float32),
        compiler_params=pltpu.CompilerParams(
            dimension_semantics=("parallel",),
        ),
    )(A_hat, E)


# speedup vs baseline: 1.0060x; 1.0060x over previous
import jax
import jax.numpy as jnp
from jax.experimental import pallas as pl
from jax.experimental.pallas import tpu as pltpu

_BM = 200


def _gcn_block(a_ref, e_ref, o_ref):
    o_ref[...] = jnp.dot(a_ref[...], e_ref[...],
                         preferred_element_type=jnp.float32)


def kernel(A_hat, E):
    n, k = A_hat.shape
    d = E.shape[1]
    bm = _BM if n % _BM == 0 else n
    return pl.pallas_call(
        _gcn_block,
        grid=(n // bm,),
        in_specs=[
            pl.BlockSpec((bm, k), lambda i: (i, 0)),
            pl.BlockSpec((k, d), lambda i: (0, 0)),
        ],
        out_specs=pl.BlockSpec((bm, d), lambda i: (i, 0)),
        out_shape=jax.ShapeDtypeStruct((n, d), jnp.float32),
        compiler_params=pltpu.CompilerParams(
            dimension_semantics=("parallel",),
        ),
    )(A_hat, E)
